# Initial kernel scaffold; baseline (speedup 1.0000x reference)
import jax
import jax.numpy as jnp
from jax.experimental import pallas as pl


def kernel(x_s, x_t, edge_index, edge_attr, x_u, W1, b1, W2, b2, U1, c1, U2, c2, gamma, beta):
    raise NotImplementedError("write your pallas kernel here")


# speedup vs baseline: 2.8851x; 2.8851x over previous
import functools

import jax
import jax.numpy as jnp
from jax import lax
from jax.experimental import pallas as pl
from jax.experimental.pallas import tpu as pltpu
from jax.experimental.pallas import tpu_sc as plsc

N_NODES = 10000
N_EDGES = 320000
D = 128
MSG_D = 2 * D
UPD_D = 4 * D
LEAKY_SLOPE = 0.01

SC_CORES = 2
SC_SUBCORES = 16
NW = SC_CORES * SC_SUBCORES

CH = 128
NCHUNK = N_EDGES // CH

def _sc_mesh():
    return plsc.VectorSubcoreMesh(
        core_axis_name="c", subcore_axis_name="s",
        num_cores=SC_CORES, num_subcores=SC_SUBCORES)


@functools.cache
def _sc_gather_fn():
    @functools.partial(
        pl.kernel,
        out_type=jax.ShapeDtypeStruct((N_EDGES, D), jnp.float32),
        mesh=_sc_mesh(),
        scratch_types=[
            pltpu.VMEM((CH,), jnp.int32),
            pltpu.VMEM((CH, D), jnp.float32),
            pltpu.SemaphoreType.DMA,
        ],
    )
    def _sc_gather(xs_hbm, src_hbm, out_hbm, idx_v, rows_v, sem):
        wid = lax.axis_index("s") * SC_CORES + lax.axis_index("c")

        @pl.loop(wid, NCHUNK, step=NW)
        def _(k):
            base = k * CH
            pltpu.sync_copy(src_hbm.at[pl.ds(base, CH)], idx_v)
            pltpu.async_copy(xs_hbm.at[idx_v], rows_v, sem).wait()
            pltpu.sync_copy(rows_v, out_hbm.at[pl.ds(base, CH)])

    return _sc_gather


@functools.cache
def _sc_scatter_add_fn():
    @functools.partial(
        pl.kernel,
        out_type=jax.ShapeDtypeStruct((N_NODES, MSG_D), jnp.float32),
        mesh=_sc_mesh(),
        scratch_types=[
            pltpu.VMEM((CH,), jnp.int32),
            pltpu.VMEM((CH, D), jnp.float32),
            pltpu.VMEM_SHARED((N_NODES, D), jnp.float32),
        ],
    )
    def _sc_scatter_add(h_hbm, tgt_hbm, zeros_hbm, out_hbm, idx_v, rows_v, acc_sh):
        cid = lax.axis_index("c")
        sid = lax.axis_index("s")
        col = cid * D

        @pl.when(sid == 0)
        def _():
            pltpu.sync_copy(zeros_hbm, acc_sh)

        plsc.subcore_barrier()

        @pl.loop(sid, NCHUNK, step=SC_SUBCORES)
        def _(k):
            base = k * CH
            pltpu.sync_copy(tgt_hbm.at[pl.ds(base, CH)], idx_v)
            pltpu.sync_copy(h_hbm.at[pl.ds(base, CH), pl.ds(col, D)], rows_v)
            pltpu.sync_copy(rows_v, acc_sh.at[idx_v], add=True)

        plsc.subcore_barrier()

        @pl.when(sid == 0)
        def _():
            pltpu.sync_copy(acc_sh, out_hbm.at[:, pl.ds(col, D)])

    return _sc_scatter_add


EB = 4000


def _msg_body(g_ref, e_ref, w1at_ref, w1bt_ref, b1_ref, w2t_ref, b2_ref, o_ref):
    h = jnp.dot(g_ref[...], w1at_ref[...], preferred_element_type=jnp.float32)
    h = h + jnp.dot(e_ref[...], w1bt_ref[...], preferred_element_type=jnp.float32)
    h = h + b1_ref[...]
    h = jnp.where(h > 0, h, LEAKY_SLOPE * h)
    o_ref[...] = (
        jnp.dot(h, w2t_ref[...], preferred_element_type=jnp.float32) + b2_ref[...])


def _tc_message(g, edge_attr, w1at, w1bt, b1r, w2t, b2r):
    full = lambda shape: pl.BlockSpec(shape, lambda i: (0, 0))
    return pl.pallas_call(
        _msg_body,
        grid=(N_EDGES // EB,),
        in_specs=[
            pl.BlockSpec((EB, D), lambda i: (i, 0)),
            pl.BlockSpec((EB, D), lambda i: (i, 0)),
            full((D, MSG_D)),
            full((D, MSG_D)),
            full((1, MSG_D)),
            full((MSG_D, MSG_D)),
            full((1, MSG_D)),
        ],
        out_specs=pl.BlockSpec((EB, MSG_D), lambda i: (i, 0)),
        out_shape=jax.ShapeDtypeStruct((N_EDGES, MSG_D), jnp.float32),
    )(g, edge_attr, w1at, w1bt, b1r, w2t, b2r)


def _upd_body(xt_ref, agg_ref, xu_ref, u1at_ref, u1bt_ref, u1ct_ref, c1_ref,
              u2t_ref, c2_ref, gamma_ref, beta_ref, o_ref):
    t = jnp.dot(xt_ref[...], u1at_ref[...], preferred_element_type=jnp.float32)
    t = t + jnp.dot(agg_ref[...], u1bt_ref[...], preferred_element_type=jnp.float32)
    t = t + jnp.dot(xu_ref[...], u1ct_ref[...], preferred_element_type=jnp.float32)
    t = t + c1_ref[...]
    t = jnp.where(t > 0, t, LEAKY_SLOPE * t)
    u = jnp.dot(t, u2t_ref[...], preferred_element_type=jnp.float32) + c2_ref[...]
    mean = jnp.mean(u, axis=0, keepdims=True)
    var = jnp.mean((u - mean) ** 2, axis=0, keepdims=True)
    o_ref[...] = (u - mean) * lax.rsqrt(var + 1e-5) * gamma_ref[...] + beta_ref[...]


def _tc_update(x_t, agg, xur, u1at, u1bt, u1ct, c1r, u2t, c2r, gammar, betar):
    return pl.pallas_call(
        _upd_body,
        out_shape=jax.ShapeDtypeStruct((N_NODES, D), jnp.float32),
    )(x_t, agg, xur, u1at, u1bt, u1ct, c1r, u2t, c2r, gammar, betar)


def kernel(x_s, x_t, edge_index, edge_attr, x_u, W1, b1, W2, b2,
           U1, c1, U2, c2, gamma, beta):
    src = edge_index[0]
    tgt = edge_index[1]

    w1t = W1.T
    w1at, w1bt = w1t[:D], w1t[D:]
    u1t = U1.T
    u1at, u1bt, u1ct = u1t[:D], u1t[D:D + MSG_D], u1t[D + MSG_D:]
    u2t = U2.T

    g = _sc_gather_fn()(x_s, src)
    h = _tc_message(g, edge_attr, w1at, w1bt, b1.reshape(1, -1), W2.T,
                    b2.reshape(1, -1))
    zeros = jnp.zeros((N_NODES, D), jnp.float32)
    agg = _sc_scatter_add_fn()(h, tgt, zeros)
    out = _tc_update(x_t, agg, x_u.reshape(1, -1), u1at, u1bt, u1ct,
                     c1.reshape(1, -1), u2t, c2.reshape(1, -1),
                     gamma.reshape(1, -1), beta.reshape(1, -1))
    return out


# speedup vs baseline: 3.8767x; 1.3437x over previous
import functools

import jax
import jax.numpy as jnp
from jax import lax
from jax.experimental import pallas as pl
from jax.experimental.pallas import tpu as pltpu
from jax.experimental.pallas import tpu_sc as plsc

N_NODES = 10000
N_EDGES = 320000
D = 128
MSG_D = 2 * D
UPD_D = 4 * D
LEAKY_SLOPE = 0.01

SC_CORES = 2
SC_SUBCORES = 16
NW = SC_CORES * SC_SUBCORES

CH = 128
NCHUNK = N_EDGES // CH
STRIPE = 624
STRIPE_REM = N_NODES - STRIPE * SC_SUBCORES


def _sc_mesh():
    return plsc.VectorSubcoreMesh(
        core_axis_name="c", subcore_axis_name="s",
        num_cores=SC_CORES, num_subcores=SC_SUBCORES)


@functools.cache
def _sc_gather_fn():
    jmax = -(-NCHUNK // NW)
    njp = ((jmax + 2 + 1) // 2) * 2

    @functools.partial(
        pl.kernel,
        out_type=jax.ShapeDtypeStruct((N_EDGES, D), jnp.float32),
        mesh=_sc_mesh(),
        scratch_types=[
            pltpu.VMEM((CH,), jnp.int32), pltpu.VMEM((CH,), jnp.int32),
            pltpu.VMEM((CH, D), jnp.float32), pltpu.VMEM((CH, D), jnp.float32),
            pltpu.SemaphoreType.DMA, pltpu.SemaphoreType.DMA,
            pltpu.SemaphoreType.DMA, pltpu.SemaphoreType.DMA,
            pltpu.SemaphoreType.DMA, pltpu.SemaphoreType.DMA,
        ],
    )
    def _sc_gather(xs_hbm, src_hbm, out_hbm, idx0, idx1, rows0, rows1,
                   si0, si1, sg0, sg1, so0, so1):
        wid = lax.axis_index("s") * SC_CORES + lax.axis_index("c")
        idx = (idx0, idx1)
        rows = (rows0, rows1)
        si = (si0, si1)
        sg = (sg0, sg1)
        so = (so0, so1)

        def chunk(j):
            return wid + j * NW

        pltpu.async_copy(src_hbm.at[pl.ds(chunk(0) * CH, CH)], idx[0], si[0])

        @pl.loop(0, njp, step=2)
        def _(j0):
            for b in range(2):
                nb = 1 - b
                j = j0 + b
                k = chunk(j)
                kp = chunk(j - 1)
                kn = chunk(j + 1)

                @pl.when(jnp.logical_and(j >= 1, kp < NCHUNK))
                def _():
                    pltpu.make_async_copy(
                        xs_hbm.at[pl.ds(0, CH)], rows[nb], sg[nb]).wait()
                    pltpu.async_copy(
                        rows[nb], out_hbm.at[pl.ds(kp * CH, CH)], so[nb])

                @pl.when(jnp.logical_and(j >= 2, chunk(j - 2) < NCHUNK))
                def _():
                    pltpu.make_async_copy(
                        rows[b], out_hbm.at[pl.ds(0, CH)], so[b]).wait()

                @pl.when(kn < NCHUNK)
                def _():
                    pltpu.async_copy(
                        src_hbm.at[pl.ds(kn * CH, CH)], idx[nb], si[nb])

                @pl.when(k < NCHUNK)
                def _():
                    pltpu.make_async_copy(
                        src_hbm.at[pl.ds(0, CH)], idx[b], si[b]).wait()
                    pltpu.async_copy(xs_hbm.at[idx[b]], rows[b], sg[b])

    return _sc_gather


@functools.cache
def _sc_scatter_add_fn():
    jmax = -(-NCHUNK // SC_SUBCORES)
    njp = ((jmax + 2 + 1) // 2) * 2

    @functools.partial(
        pl.kernel,
        out_type=jax.ShapeDtypeStruct((N_NODES, MSG_D), jnp.float32),
        mesh=_sc_mesh(),
        scratch_types=[
            pltpu.VMEM((CH,), jnp.int32), pltpu.VMEM((CH,), jnp.int32),
            pltpu.VMEM((CH, D), jnp.float32), pltpu.VMEM((CH, D), jnp.float32),
            pltpu.VMEM_SHARED((N_NODES, D), jnp.float32),
            pltpu.SemaphoreType.DMA, pltpu.SemaphoreType.DMA,
            pltpu.SemaphoreType.DMA, pltpu.SemaphoreType.DMA,
            pltpu.SemaphoreType.DMA, pltpu.SemaphoreType.DMA,
        ],
    )
    def _sc_scatter_add(h_hbm, tgt_hbm, zeros_hbm, out_hbm, idx0, idx1,
                        rows0, rows1, acc_sh, si0, si1, sr0, sr1, sa0, sa1):
        cid = lax.axis_index("c")
        sid = lax.axis_index("s")
        col = cid * D
        idx = (idx0, idx1)
        rows = (rows0, rows1)
        si = (si0, si1)
        sr = (sr0, sr1)
        sa = (sa0, sa1)

        stripe = sid * STRIPE
        pltpu.sync_copy(zeros_hbm.at[pl.ds(stripe, STRIPE)],
                        acc_sh.at[pl.ds(stripe, STRIPE)])

        @pl.when(sid == 0)
        def _():
            rem = STRIPE * SC_SUBCORES
            pltpu.sync_copy(zeros_hbm.at[pl.ds(rem, STRIPE_REM)],
                            acc_sh.at[pl.ds(rem, STRIPE_REM)])

        plsc.subcore_barrier()

        def chunk(j):
            return sid + j * SC_SUBCORES

        @pl.loop(0, njp, step=2)
        def _(j0):
            for b in range(2):
                j = j0 + b
                k = chunk(j)

                @pl.when(jnp.logical_and(j >= 2, chunk(j - 2) < NCHUNK))
                def _():
                    pltpu.make_async_copy(
                        rows[b], acc_sh.at[pl.ds(0, CH)], sa[b]).wait()

                @pl.when(k < NCHUNK)
                def _():
                    pltpu.async_copy(
                        tgt_hbm.at[pl.ds(k * CH, CH)], idx[b], si[b])
                    pltpu.async_copy(
                        h_hbm.at[pl.ds(k * CH, CH), pl.ds(col, D)],
                        rows[b], sr[b])
                    pltpu.make_async_copy(
                        tgt_hbm.at[pl.ds(0, CH)], idx[b], si[b]).wait()
                    pltpu.make_async_copy(
                        h_hbm.at[pl.ds(0, CH), pl.ds(col, D)],
                        rows[b], sr[b]).wait()
                    pltpu.async_copy(rows[b], acc_sh.at[idx[b]], sa[b],
                                     add=True)

        plsc.subcore_barrier()
        pltpu.sync_copy(acc_sh.at[pl.ds(stripe, STRIPE)],
                        out_hbm.at[pl.ds(stripe, STRIPE), pl.ds(col, D)])

        @pl.when(sid == 0)
        def _():
            rem = STRIPE * SC_SUBCORES
            pltpu.sync_copy(acc_sh.at[pl.ds(rem, STRIPE_REM)],
                            out_hbm.at[pl.ds(rem, STRIPE_REM), pl.ds(col, D)])

    return _sc_scatter_add


EB = 4000


def _msg_body(g_ref, e_ref, w1at_ref, w1bt_ref, b1_ref, w2t_ref, b2_ref, o_ref):
    h = jnp.dot(g_ref[...], w1at_ref[...], preferred_element_type=jnp.float32)
    h = h + jnp.dot(e_ref[...], w1bt_ref[...], preferred_element_type=jnp.float32)
    h = h + b1_ref[...]
    h = jnp.where(h > 0, h, LEAKY_SLOPE * h)
    o_ref[...] = (
        jnp.dot(h, w2t_ref[...], preferred_element_type=jnp.float32) + b2_ref[...])


def _tc_message(g, edge_attr, w1at, w1bt, b1r, w2t, b2r):
    full = lambda shape: pl.BlockSpec(shape, lambda i: (0, 0))
    return pl.pallas_call(
        _msg_body,
        grid=(N_EDGES // EB,),
        in_specs=[
            pl.BlockSpec((EB, D), lambda i: (i, 0)),
            pl.BlockSpec((EB, D), lambda i: (i, 0)),
            full((D, MSG_D)),
            full((D, MSG_D)),
            full((1, MSG_D)),
            full((MSG_D, MSG_D)),
            full((1, MSG_D)),
        ],
        out_specs=pl.BlockSpec((EB, MSG_D), lambda i: (i, 0)),
        out_shape=jax.ShapeDtypeStruct((N_EDGES, MSG_D), jnp.float32),
    )(g, edge_attr, w1at, w1bt, b1r, w2t, b2r)


def _upd_body(xt_ref, agg_ref, xu_ref, u1at_ref, u1bt_ref, u1ct_ref, c1_ref,
              u2t_ref, c2_ref, gamma_ref, beta_ref, o_ref):
    t = jnp.dot(xt_ref[...], u1at_ref[...], preferred_element_type=jnp.float32)
    t = t + jnp.dot(agg_ref[...], u1bt_ref[...], preferred_element_type=jnp.float32)
    t = t + jnp.dot(xu_ref[...], u1ct_ref[...], preferred_element_type=jnp.float32)
    t = t + c1_ref[...]
    t = jnp.where(t > 0, t, LEAKY_SLOPE * t)
    u = jnp.dot(t, u2t_ref[...], preferred_element_type=jnp.float32) + c2_ref[...]
    mean = jnp.mean(u, axis=0, keepdims=True)
    var = jnp.mean((u - mean) ** 2, axis=0, keepdims=True)
    o_ref[...] = (u - mean) * lax.rsqrt(var + 1e-5) * gamma_ref[...] + beta_ref[...]


def _tc_update(x_t, agg, xur, u1at, u1bt, u1ct, c1r, u2t, c2r, gammar, betar):
    return pl.pallas_call(
        _upd_body,
        out_shape=jax.ShapeDtypeStruct((N_NODES, D), jnp.float32),
    )(x_t, agg, xur, u1at, u1bt, u1ct, c1r, u2t, c2r, gammar, betar)


def kernel(x_s, x_t, edge_index, edge_attr, x_u, W1, b1, W2, b2,
           U1, c1, U2, c2, gamma, beta):
    src = edge_index[0]
    tgt = edge_index[1]

    w1t = W1.T
    w1at, w1bt = w1t[:D], w1t[D:]
    u1t = U1.T
    u1at, u1bt, u1ct = u1t[:D], u1t[D:D + MSG_D], u1t[D + MSG_D:]
    u2t = U2.T

    g = _sc_gather_fn()(x_s, src)
    h = _tc_message(g, edge_attr, w1at, w1bt, b1.reshape(1, -1), W2.T,
                    b2.reshape(1, -1))
    zeros = jnp.zeros((N_NODES, D), jnp.float32)
    agg = _sc_scatter_add_fn()(h, tgt, zeros)
    out = _tc_update(x_t, agg, x_u.reshape(1, -1), u1at, u1bt, u1ct,
                     c1.reshape(1, -1), u2t, c2.reshape(1, -1),
                     gamma.reshape(1, -1), beta.reshape(1, -1))
    return out
